# baseline (speedup 1.0000x reference)
import dataclasses
import functools

import jax
import jax.numpy as jnp
from jax import lax
from jax.experimental import pallas as pl
from jax.experimental.pallas import tpu as pltpu
from jax.experimental.pallas import tpu_sc as plsc

NUM_GROUP = 256
GROUP_SIZE = 32
_INF = float("inf")


def _fps_body(x_ref, y_ref, z_ref, cx_ref, cy_ref, cz_ref, dists_ref):
    B, N = x_ref.shape
    G = cx_ref.shape[1]
    lane = lax.broadcasted_iota(jnp.int32, (B, N), 1)
    glane = lax.broadcasted_iota(jnp.int32, (B, G), 1)

    x = x_ref[...]
    y = y_ref[...]
    z = z_ref[...]

    lx0 = x[:, 0:1]
    ly0 = y[:, 0:1]
    lz0 = z[:, 0:1]
    cx_ref[...] = jnp.where(glane == 0, lx0, 0.0)
    cy_ref[...] = jnp.where(glane == 0, ly0, 0.0)
    cz_ref[...] = jnp.where(glane == 0, lz0, 0.0)
    dists_ref[...] = jnp.full((B, N), 1e10, jnp.float32)

    def body(i, carry):
        lx, ly, lz = carry
        dx = x - lx
        dy = y - ly
        dz = z - lz
        d = dx * dx + dy * dy + dz * dz
        dn = jnp.minimum(dists_ref[...], d)
        dists_ref[...] = dn
        m = jnp.max(dn, axis=1, keepdims=True)
        nxt = jnp.min(jnp.where(dn == m, lane, N), axis=1, keepdims=True)
        msk = lane == nxt
        nlx = jnp.sum(jnp.where(msk, x, 0.0), axis=1, keepdims=True)
        nly = jnp.sum(jnp.where(msk, y, 0.0), axis=1, keepdims=True)
        nlz = jnp.sum(jnp.where(msk, z, 0.0), axis=1, keepdims=True)
        cx_ref[...] = jnp.where(glane == i, nlx, cx_ref[...])
        cy_ref[...] = jnp.where(glane == i, nly, cy_ref[...])
        cz_ref[...] = jnp.where(glane == i, nlz, cz_ref[...])
        return (nlx, nly, nlz)

    lax.fori_loop(1, G, body, (lx0, ly0, lz0), unroll=False)


def _fps_centers(x, y, z):
    B, N = x.shape
    out = jax.ShapeDtypeStruct((B, NUM_GROUP), jnp.float32)
    return pl.pallas_call(
        _fps_body,
        out_shape=[out, out, out],
        scratch_shapes=[pltpu.VMEM((B, N), jnp.float32)],
    )(x, y, z)


def _sc_mesh():
    return plsc.VectorSubcoreMesh(core_axis_name="c", subcore_axis_name="s")


def _merge32(K0, I0, K1, I1, C, CI):
    sC, sCI = plsc.sort_key_val(C, CI)
    rC = lax.rev(sC, (0,))
    rCI = lax.rev(sCI, (0,))
    selA = K1 <= rC
    mlo = jnp.where(selA, K1, rC)
    mli = jnp.where(selA, I1, rCI)
    T, TI = plsc.sort_key_val(mlo, mli)
    rT = lax.rev(T, (0,))
    rTI = lax.rev(TI, (0,))
    selB = K0 <= rT
    l = jnp.where(selB, K0, rT)
    li = jnp.where(selB, I0, rTI)
    h = jnp.where(selB, rT, K0)
    hi = jnp.where(selB, rTI, I0)
    K0, I0 = plsc.sort_key_val(l, li)
    K1, I1 = plsc.sort_key_val(h, hi)
    return K0, I0, K1, I1


def _round_bf16(v):
    u = plsc.bitcast(v, jnp.int32)
    r = (u + 0x7FFF + (lax.shift_right_logical(u, 16) & 1)) & -65536
    return plsc.bitcast(r, jnp.float32)


def _knn_sc_body(x_hbm, y_hbm, z_hbm, cx_hbm, cy_hbm, cz_hbm, out_hbm,
                 xv, yv, zv, xb, yb, zb, xn2v, d2buf, candi,
                 cxv, cyv, czv, neigh_v):
    N = xv.shape[0]
    GSH = cxv.shape[0]
    NBLK = 32
    CPB = N // (NBLK * 16)

    NSH = 32 // x_hbm.shape[0]
    wid = lax.axis_index("s") * 2 + lax.axis_index("c")
    b = wid // NSH
    q = wid % NSH

    pltpu.sync_copy(x_hbm.at[b], xv)
    pltpu.sync_copy(y_hbm.at[b], yv)
    pltpu.sync_copy(z_hbm.at[b], zv)
    pltpu.sync_copy(cx_hbm.at[b, pl.ds(q * GSH, GSH)], cxv)
    pltpu.sync_copy(cy_hbm.at[b, pl.ds(q * GSH, GSH)], cyv)
    pltpu.sync_copy(cz_hbm.at[b, pl.ds(q * GSH, GSH)], czv)

    iota16 = lax.iota(jnp.int32, 16)
    zf16 = jnp.zeros((16,), jnp.float32)
    zi16 = jnp.zeros((16,), jnp.int32)

    def pre_body(i, _):
        off = i * 16
        xc = xv[pl.ds(off, 16)]
        yc = yv[pl.ds(off, 16)]
        zc = zv[pl.ds(off, 16)]
        xn2v[pl.ds(off, 16)] = (xc * xc + yc * yc) + zc * zc
        xb[pl.ds(off, 16)] = _round_bf16(xc)
        yb[pl.ds(off, 16)] = _round_bf16(yc)
        zb[pl.ds(off, 16)] = _round_bf16(zc)
        return 0

    lax.fori_loop(0, N // 16, pre_body, 0)

    def _select_store(dref, U, g):
        Uv = zf16 + U

        def fil_body(i, cntv):
            d2v = dref[pl.ds(i * 16, 16)]
            m = d2v <= Uv
            pos = cntv + plsc.cumsum(m.astype(jnp.int32))
            plsc.store_scatter(candi, [pos], iota16 + i * 16, mask=m)
            return cntv + plsc.all_reduce_population_count(m)

        cntv = lax.fori_loop(0, N // 16, fil_body, zi16 - 1, unroll=8)

        cntv = cntv + 1
        cnt = jnp.max(cntv)
        nch = (cnt + 15) // 16

        def mg_body(j, carry):
            K0, I0, K1, I1 = carry
            valid = (iota16 + j * 16) < cntv
            CI = jnp.where(valid, candi[pl.ds(j * 16, 16)], 0)
            C = jnp.where(valid, plsc.load_gather(dref, [CI]), _INF)
            return _merge32(K0, I0, K1, I1, C, CI)

        K0, I0, K1, I1 = lax.fori_loop(
            0, nch, mg_body,
            (zf16 + _INF, zi16, zf16 + _INF, zi16))

        base = g * 96
        neigh_v[pl.ds(base, 16)] = plsc.load_gather(xv, [I0])
        neigh_v[pl.ds(base + 16, 16)] = plsc.load_gather(xv, [I1])
        neigh_v[pl.ds(base + 32, 16)] = plsc.load_gather(yv, [I0])
        neigh_v[pl.ds(base + 48, 16)] = plsc.load_gather(yv, [I1])
        neigh_v[pl.ds(base + 64, 16)] = plsc.load_gather(zv, [I0])
        neigh_v[pl.ds(base + 80, 16)] = plsc.load_gather(zv, [I1])

    @pl.loop(0, GSH)
    def _center(g):
        gv = zi16 + g
        cxs = plsc.load_gather(cxv, [gv])
        cys = plsc.load_gather(cyv, [gv])
        czs = plsc.load_gather(czv, [gv])
        cn2 = (cxs * cxs + cys * cys) + czs * czs
        cxb = _round_bf16(cxs)
        cyb = _round_bf16(cys)
        czb = _round_bf16(czs)

        def blk_body(blk, U):
            bmin = zf16 + _INF
            base = blk * (CPB * 16)
            for cc in range(CPB):
                off = base + cc * 16
                dot = (xb[pl.ds(off, 16)] * cxb
                       + yb[pl.ds(off, 16)] * cyb) + zb[pl.ds(off, 16)] * czb
                d2 = (cn2 + xn2v[pl.ds(off, 16)]) - (dot + dot)
                d2buf[pl.ds(off, 16)] = d2
                bmin = jnp.minimum(bmin, d2)
            return jnp.maximum(U, jnp.min(bmin))

        U = lax.fori_loop(0, NBLK, blk_body, -_INF)
        _select_store(d2buf, U, g)

    pltpu.sync_copy(neigh_v, out_hbm.at[b, pl.ds(q * GSH * 96, GSH * 96)])


def _knn_group_sc(x, y, z, cx, cy, cz):
    B, N = x.shape
    G = cx.shape[1]
    GSH = G // (32 // B)
    cp = pltpu.CompilerParams()
    if "needs_layout_passes" in pltpu.CompilerParams.__dataclass_fields__:
        cp = dataclasses.replace(cp, needs_layout_passes=False)
    kfn = pl.kernel(
        _knn_sc_body,
        out_type=jax.ShapeDtypeStruct((B, G * 96), jnp.float32),
        mesh=_sc_mesh(),
        scratch_types=[
            pltpu.VMEM((N,), jnp.float32),
            pltpu.VMEM((N,), jnp.float32),
            pltpu.VMEM((N,), jnp.float32),
            pltpu.VMEM((N,), jnp.float32),
            pltpu.VMEM((N,), jnp.float32),
            pltpu.VMEM((N,), jnp.float32),
            pltpu.VMEM((N,), jnp.float32),
            pltpu.VMEM((N,), jnp.float32),
            pltpu.VMEM((N,), jnp.int32),
            pltpu.VMEM((GSH,), jnp.float32),
            pltpu.VMEM((GSH,), jnp.float32),
            pltpu.VMEM((GSH,), jnp.float32),
            pltpu.VMEM((GSH * 96,), jnp.float32),
        ],
        compiler_params=cp,
    )
    return kfn(x, y, z, cx, cy, cz)


def kernel(xyz, pose):
    B, N, _ = xyz.shape
    R = pose[:, :, :3]
    t = pose[:, :, 3]
    xyz_t = jnp.einsum('bij,bnj->bni', R, xyz) + t[:, None, :]
    planes = xyz_t.transpose(2, 0, 1)
    x, y, z = planes[0], planes[1], planes[2]

    H = B // 2
    cxa, cya, cza = _fps_centers(x[:H], y[:H], z[:H])
    cxb, cyb, czb = _fps_centers(x[H:], y[H:], z[H:])
    flata = _knn_group_sc(x[:H], y[:H], z[:H], cxa, cya, cza)
    flatb = _knn_group_sc(x[H:], y[H:], z[H:], cxb, cyb, czb)
    cx = jnp.concatenate([cxa, cxb], axis=0)
    cy = jnp.concatenate([cya, cyb], axis=0)
    cz = jnp.concatenate([cza, czb], axis=0)
    center = jnp.stack([cx, cy, cz], axis=-1)
    flat = jnp.concatenate([flata, flatb], axis=0)
    neighborhood = flat.reshape(B, NUM_GROUP, 3, GROUP_SIZE).transpose(0, 1, 3, 2)
    return (neighborhood, center)


# speedup vs baseline: 1.0088x; 1.0088x over previous
import dataclasses
import functools

import jax
import jax.numpy as jnp
from jax import lax
from jax.experimental import pallas as pl
from jax.experimental.pallas import tpu as pltpu
from jax.experimental.pallas import tpu_sc as plsc

NUM_GROUP = 256
GROUP_SIZE = 32
_INF = float("inf")


def _fps_body(x_ref, y_ref, z_ref, cx_ref, cy_ref, cz_ref, dists_ref):
    B, N = x_ref.shape
    G = cx_ref.shape[1]
    lane = lax.broadcasted_iota(jnp.int32, (B, N), 1)
    glane = lax.broadcasted_iota(jnp.int32, (B, G), 1)

    x = x_ref[...]
    y = y_ref[...]
    z = z_ref[...]

    lx0 = x[:, 0:1]
    ly0 = y[:, 0:1]
    lz0 = z[:, 0:1]
    cx_ref[...] = jnp.where(glane == 0, lx0, 0.0)
    cy_ref[...] = jnp.where(glane == 0, ly0, 0.0)
    cz_ref[...] = jnp.where(glane == 0, lz0, 0.0)
    dists_ref[...] = jnp.full((B, N), 1e10, jnp.float32)

    def body(i, carry):
        lx, ly, lz = carry
        dx = x - lx
        dy = y - ly
        dz = z - lz
        d = dx * dx + dy * dy + dz * dz
        dn = jnp.minimum(dists_ref[...], d)
        dists_ref[...] = dn
        m = jnp.max(dn, axis=1, keepdims=True)
        nxt = jnp.min(jnp.where(dn == m, lane, N), axis=1, keepdims=True)
        msk = lane == nxt
        nlx = jnp.sum(jnp.where(msk, x, 0.0), axis=1, keepdims=True)
        nly = jnp.sum(jnp.where(msk, y, 0.0), axis=1, keepdims=True)
        nlz = jnp.sum(jnp.where(msk, z, 0.0), axis=1, keepdims=True)
        cx_ref[...] = jnp.where(glane == i, nlx, cx_ref[...])
        cy_ref[...] = jnp.where(glane == i, nly, cy_ref[...])
        cz_ref[...] = jnp.where(glane == i, nlz, cz_ref[...])
        return (nlx, nly, nlz)

    lax.fori_loop(1, G, body, (lx0, ly0, lz0), unroll=False)


def _fps_centers(x, y, z):
    B, N = x.shape
    out = jax.ShapeDtypeStruct((B, NUM_GROUP), jnp.float32)
    return pl.pallas_call(
        _fps_body,
        out_shape=[out, out, out],
        scratch_shapes=[pltpu.VMEM((B, N), jnp.float32)],
    )(x, y, z)


def _sc_mesh():
    return plsc.VectorSubcoreMesh(core_axis_name="c", subcore_axis_name="s")


def _merge32(K0, I0, K1, I1, C, CI):
    sC, sCI = plsc.sort_key_val(C, CI)
    rC = lax.rev(sC, (0,))
    rCI = lax.rev(sCI, (0,))
    selA = K1 <= rC
    mlo = jnp.where(selA, K1, rC)
    mli = jnp.where(selA, I1, rCI)
    T, TI = plsc.sort_key_val(mlo, mli)
    rT = lax.rev(T, (0,))
    rTI = lax.rev(TI, (0,))
    selB = K0 <= rT
    l = jnp.where(selB, K0, rT)
    li = jnp.where(selB, I0, rTI)
    h = jnp.where(selB, rT, K0)
    hi = jnp.where(selB, rTI, I0)
    K0, I0 = plsc.sort_key_val(l, li)
    K1, I1 = plsc.sort_key_val(h, hi)
    return K0, I0, K1, I1


def _round_bf16(v):
    u = plsc.bitcast(v, jnp.int32)
    r = (u + 0x7FFF + (lax.shift_right_logical(u, 16) & 1)) & -65536
    return plsc.bitcast(r, jnp.float32)


def _knn_sc_body(x_hbm, y_hbm, z_hbm, cx_hbm, cy_hbm, cz_hbm, out_hbm,
                 xv, yv, zv, xb, yb, zb, xn2v, d2buf, candi,
                 cxv, cyv, czv, neigh_v):
    N = xv.shape[0]
    GSH = cxv.shape[0]
    NBLK = 32
    CPB = N // (NBLK * 16)

    wid = lax.axis_index("s") * 2 + lax.axis_index("c")
    b = wid // 4
    q = wid % 4

    pltpu.sync_copy(x_hbm.at[b], xv)
    pltpu.sync_copy(y_hbm.at[b], yv)
    pltpu.sync_copy(z_hbm.at[b], zv)
    pltpu.sync_copy(cx_hbm.at[b, pl.ds(q * GSH, GSH)], cxv)
    pltpu.sync_copy(cy_hbm.at[b, pl.ds(q * GSH, GSH)], cyv)
    pltpu.sync_copy(cz_hbm.at[b, pl.ds(q * GSH, GSH)], czv)

    iota16 = lax.iota(jnp.int32, 16)
    zf16 = jnp.zeros((16,), jnp.float32)
    zi16 = jnp.zeros((16,), jnp.int32)

    def pre_body(i, _):
        off = i * 16
        xc = xv[pl.ds(off, 16)]
        yc = yv[pl.ds(off, 16)]
        zc = zv[pl.ds(off, 16)]
        xn2v[pl.ds(off, 16)] = (xc * xc + yc * yc) + zc * zc
        xb[pl.ds(off, 16)] = _round_bf16(xc)
        yb[pl.ds(off, 16)] = _round_bf16(yc)
        zb[pl.ds(off, 16)] = _round_bf16(zc)
        return 0

    lax.fori_loop(0, N // 16, pre_body, 0)

    def _select_store(dref, U, g):
        Uv = zf16 + U

        def fil_body(i, cntv):
            d2v = dref[pl.ds(i * 16, 16)]
            m = d2v <= Uv
            pos = cntv + plsc.cumsum(m.astype(jnp.int32))
            plsc.store_scatter(candi, [pos], iota16 + i * 16, mask=m)
            return cntv + plsc.all_reduce_population_count(m)

        cntv = lax.fori_loop(0, N // 16, fil_body, zi16 - 1, unroll=8)

        cntv = cntv + 1
        cnt = jnp.max(cntv)
        nch = (cnt + 15) // 16

        def mg_body(j, carry):
            K0, I0, K1, I1 = carry
            valid = (iota16 + j * 16) < cntv
            CI = jnp.where(valid, candi[pl.ds(j * 16, 16)], 0)
            C = jnp.where(valid, plsc.load_gather(dref, [CI]), _INF)
            return _merge32(K0, I0, K1, I1, C, CI)

        K0, I0, K1, I1 = lax.fori_loop(
            0, nch, mg_body,
            (zf16 + _INF, zi16, zf16 + _INF, zi16))

        base = g * 96
        neigh_v[pl.ds(base, 16)] = plsc.load_gather(xv, [I0])
        neigh_v[pl.ds(base + 16, 16)] = plsc.load_gather(xv, [I1])
        neigh_v[pl.ds(base + 32, 16)] = plsc.load_gather(yv, [I0])
        neigh_v[pl.ds(base + 48, 16)] = plsc.load_gather(yv, [I1])
        neigh_v[pl.ds(base + 64, 16)] = plsc.load_gather(zv, [I0])
        neigh_v[pl.ds(base + 80, 16)] = plsc.load_gather(zv, [I1])

    @pl.loop(0, GSH)
    def _center(g):
        gv = zi16 + g
        cxs = plsc.load_gather(cxv, [gv])
        cys = plsc.load_gather(cyv, [gv])
        czs = plsc.load_gather(czv, [gv])
        cn2 = (cxs * cxs + cys * cys) + czs * czs
        cxb = _round_bf16(cxs)
        cyb = _round_bf16(cys)
        czb = _round_bf16(czs)

        def blk_body(blk, U):
            bmin = zf16 + _INF
            base = blk * (CPB * 16)
            for cc in range(CPB):
                off = base + cc * 16
                dot = (xb[pl.ds(off, 16)] * cxb
                       + yb[pl.ds(off, 16)] * cyb) + zb[pl.ds(off, 16)] * czb
                d2 = (cn2 + xn2v[pl.ds(off, 16)]) - (dot + dot)
                d2buf[pl.ds(off, 16)] = d2
                bmin = jnp.minimum(bmin, d2)
            return jnp.maximum(U, jnp.min(bmin))

        U = lax.fori_loop(0, NBLK, blk_body, -_INF)
        _select_store(d2buf, U, g)

    pltpu.sync_copy(neigh_v, out_hbm.at[b, pl.ds(q * GSH * 96, GSH * 96)])


def _knn_group_sc(x, y, z, cx, cy, cz):
    B, N = x.shape
    G = cx.shape[1]
    GSH = G // 4
    cp = pltpu.CompilerParams()
    if "needs_layout_passes" in pltpu.CompilerParams.__dataclass_fields__:
        cp = dataclasses.replace(cp, needs_layout_passes=False)
    kfn = pl.kernel(
        _knn_sc_body,
        out_type=jax.ShapeDtypeStruct((B, G * 96), jnp.float32),
        mesh=_sc_mesh(),
        scratch_types=[
            pltpu.VMEM((N,), jnp.float32),
            pltpu.VMEM((N,), jnp.float32),
            pltpu.VMEM((N,), jnp.float32),
            pltpu.VMEM((N,), jnp.float32),
            pltpu.VMEM((N,), jnp.float32),
            pltpu.VMEM((N,), jnp.float32),
            pltpu.VMEM((N,), jnp.float32),
            pltpu.VMEM((N,), jnp.float32),
            pltpu.VMEM((N,), jnp.int32),
            pltpu.VMEM((GSH,), jnp.float32),
            pltpu.VMEM((GSH,), jnp.float32),
            pltpu.VMEM((GSH,), jnp.float32),
            pltpu.VMEM((GSH * 96,), jnp.float32),
        ],
        compiler_params=cp,
    )
    return kfn(x, y, z, cx, cy, cz)


def kernel(xyz, pose):
    B, N, _ = xyz.shape
    R = pose[:, :, :3]
    t = pose[:, :, 3]
    xyz_t = jnp.einsum('bij,bnj->bni', R, xyz) + t[:, None, :]
    planes = xyz_t.transpose(2, 0, 1)
    x, y, z = planes[0], planes[1], planes[2]

    cx, cy, cz = _fps_centers(x, y, z)
    center = jnp.stack([cx, cy, cz], axis=-1)

    flat = _knn_group_sc(x, y, z, cx, cy, cz)
    neighborhood = flat.reshape(B, NUM_GROUP, 3, GROUP_SIZE).transpose(0, 1, 3, 2)
    return (neighborhood, center)


# speedup vs baseline: 1.0147x; 1.0058x over previous
import dataclasses
import functools

import jax
import jax.numpy as jnp
from jax import lax
from jax.experimental import pallas as pl
from jax.experimental.pallas import tpu as pltpu
from jax.experimental.pallas import tpu_sc as plsc

NUM_GROUP = 256
GROUP_SIZE = 32
_INF = float("inf")


def _fps_body(x_ref, y_ref, z_ref, cx_ref, cy_ref, cz_ref):
    B, N = x_ref.shape
    G = cx_ref.shape[1]
    lane = lax.broadcasted_iota(jnp.int32, (B, N), 1)
    glane = lax.broadcasted_iota(jnp.int32, (B, G), 1)

    x = x_ref[...]
    y = y_ref[...]
    z = z_ref[...]

    lx0 = x[:, 0:1]
    ly0 = y[:, 0:1]
    lz0 = z[:, 0:1]
    cx_ref[...] = jnp.where(glane == 0, lx0, 0.0)
    cy_ref[...] = jnp.where(glane == 0, ly0, 0.0)
    cz_ref[...] = jnp.where(glane == 0, lz0, 0.0)

    def body(i, carry):
        lx, ly, lz, dists = carry
        dx = x - lx
        dy = y - ly
        dz = z - lz
        d = dx * dx + dy * dy + dz * dz
        dn = jnp.minimum(dists, d)
        m = jnp.max(dn, axis=1, keepdims=True)
        nxt = jnp.min(jnp.where(dn == m, lane, N), axis=1, keepdims=True)
        msk = lane == nxt
        nlx = jnp.sum(jnp.where(msk, x, 0.0), axis=1, keepdims=True)
        nly = jnp.sum(jnp.where(msk, y, 0.0), axis=1, keepdims=True)
        nlz = jnp.sum(jnp.where(msk, z, 0.0), axis=1, keepdims=True)
        cx_ref[...] = jnp.where(glane == i, nlx, cx_ref[...])
        cy_ref[...] = jnp.where(glane == i, nly, cy_ref[...])
        cz_ref[...] = jnp.where(glane == i, nlz, cz_ref[...])
        return (nlx, nly, nlz, dn)

    lax.fori_loop(1, G, body,
                  (lx0, ly0, lz0, jnp.full((B, N), 1e10, jnp.float32)),
                  unroll=False)


def _fps_centers(x, y, z):
    B, N = x.shape
    out = jax.ShapeDtypeStruct((B, NUM_GROUP), jnp.float32)
    return pl.pallas_call(
        _fps_body,
        out_shape=[out, out, out],
    )(x, y, z)


def _sc_mesh():
    return plsc.VectorSubcoreMesh(core_axis_name="c", subcore_axis_name="s")


def _merge32(K0, I0, K1, I1, C, CI):
    sC, sCI = plsc.sort_key_val(C, CI)
    rC = lax.rev(sC, (0,))
    rCI = lax.rev(sCI, (0,))
    selA = K1 <= rC
    mlo = jnp.where(selA, K1, rC)
    mli = jnp.where(selA, I1, rCI)
    T, TI = plsc.sort_key_val(mlo, mli)
    rT = lax.rev(T, (0,))
    rTI = lax.rev(TI, (0,))
    selB = K0 <= rT
    l = jnp.where(selB, K0, rT)
    li = jnp.where(selB, I0, rTI)
    h = jnp.where(selB, rT, K0)
    hi = jnp.where(selB, rTI, I0)
    K0, I0 = plsc.sort_key_val(l, li)
    K1, I1 = plsc.sort_key_val(h, hi)
    return K0, I0, K1, I1


def _round_bf16(v):
    u = plsc.bitcast(v, jnp.int32)
    r = (u + 0x7FFF + (lax.shift_right_logical(u, 16) & 1)) & -65536
    return plsc.bitcast(r, jnp.float32)


def _knn_sc_body(x_hbm, y_hbm, z_hbm, cx_hbm, cy_hbm, cz_hbm, out_hbm,
                 xv, yv, zv, xb, yb, zb, xn2v, d2buf, candi,
                 cxv, cyv, czv, neigh_v):
    N = xv.shape[0]
    GSH = cxv.shape[0]
    NBLK = 32
    CPB = N // (NBLK * 16)

    wid = lax.axis_index("s") * 2 + lax.axis_index("c")
    b = wid // 4
    q = wid % 4

    pltpu.sync_copy(x_hbm.at[b], xv)
    pltpu.sync_copy(y_hbm.at[b], yv)
    pltpu.sync_copy(z_hbm.at[b], zv)
    pltpu.sync_copy(cx_hbm.at[b, pl.ds(q * GSH, GSH)], cxv)
    pltpu.sync_copy(cy_hbm.at[b, pl.ds(q * GSH, GSH)], cyv)
    pltpu.sync_copy(cz_hbm.at[b, pl.ds(q * GSH, GSH)], czv)

    iota16 = lax.iota(jnp.int32, 16)
    zf16 = jnp.zeros((16,), jnp.float32)
    zi16 = jnp.zeros((16,), jnp.int32)

    def pre_body(i, _):
        off = i * 16
        xc = xv[pl.ds(off, 16)]
        yc = yv[pl.ds(off, 16)]
        zc = zv[pl.ds(off, 16)]
        xn2v[pl.ds(off, 16)] = (xc * xc + yc * yc) + zc * zc
        xb[pl.ds(off, 16)] = _round_bf16(xc)
        yb[pl.ds(off, 16)] = _round_bf16(yc)
        zb[pl.ds(off, 16)] = _round_bf16(zc)
        return 0

    lax.fori_loop(0, N // 16, pre_body, 0)

    def _select_store(dref, U, g):
        Uv = zf16 + U

        def fil_body(i, cntv):
            d2v = dref[pl.ds(i * 16, 16)]
            m = d2v <= Uv
            pos = cntv + plsc.cumsum(m.astype(jnp.int32))
            plsc.store_scatter(candi, [pos], iota16 + i * 16, mask=m)
            return cntv + plsc.all_reduce_population_count(m)

        cntv = lax.fori_loop(0, N // 16, fil_body, zi16 - 1, unroll=8)

        cntv = cntv + 1
        cnt = jnp.max(cntv)
        nch = (cnt + 15) // 16

        def mg_body(j, carry):
            K0, I0, K1, I1 = carry
            valid = (iota16 + j * 16) < cntv
            CI = jnp.where(valid, candi[pl.ds(j * 16, 16)], 0)
            C = jnp.where(valid, plsc.load_gather(dref, [CI]), _INF)
            return _merge32(K0, I0, K1, I1, C, CI)

        K0, I0, K1, I1 = lax.fori_loop(
            0, nch, mg_body,
            (zf16 + _INF, zi16, zf16 + _INF, zi16))

        base = g * 96
        neigh_v[pl.ds(base, 16)] = plsc.load_gather(xv, [I0])
        neigh_v[pl.ds(base + 16, 16)] = plsc.load_gather(xv, [I1])
        neigh_v[pl.ds(base + 32, 16)] = plsc.load_gather(yv, [I0])
        neigh_v[pl.ds(base + 48, 16)] = plsc.load_gather(yv, [I1])
        neigh_v[pl.ds(base + 64, 16)] = plsc.load_gather(zv, [I0])
        neigh_v[pl.ds(base + 80, 16)] = plsc.load_gather(zv, [I1])

    @pl.loop(0, GSH)
    def _center(g):
        gv = zi16 + g
        cxs = plsc.load_gather(cxv, [gv])
        cys = plsc.load_gather(cyv, [gv])
        czs = plsc.load_gather(czv, [gv])
        cn2 = (cxs * cxs + cys * cys) + czs * czs
        cxb = _round_bf16(cxs)
        cyb = _round_bf16(cys)
        czb = _round_bf16(czs)

        def blk_body(blk, U):
            bmin = zf16 + _INF
            base = blk * (CPB * 16)
            for cc in range(CPB):
                off = base + cc * 16
                dot = (xb[pl.ds(off, 16)] * cxb
                       + yb[pl.ds(off, 16)] * cyb) + zb[pl.ds(off, 16)] * czb
                d2 = (cn2 + xn2v[pl.ds(off, 16)]) - (dot + dot)
                d2buf[pl.ds(off, 16)] = d2
                bmin = jnp.minimum(bmin, d2)
            return jnp.maximum(U, jnp.min(bmin))

        U = lax.fori_loop(0, NBLK, blk_body, -_INF)
        _select_store(d2buf, U, g)

    pltpu.sync_copy(neigh_v, out_hbm.at[b, pl.ds(q * GSH * 96, GSH * 96)])


def _knn_group_sc(x, y, z, cx, cy, cz):
    B, N = x.shape
    G = cx.shape[1]
    GSH = G // 4
    cp = pltpu.CompilerParams()
    if "needs_layout_passes" in pltpu.CompilerParams.__dataclass_fields__:
        cp = dataclasses.replace(cp, needs_layout_passes=False)
    kfn = pl.kernel(
        _knn_sc_body,
        out_type=jax.ShapeDtypeStruct((B, G * 96), jnp.float32),
        mesh=_sc_mesh(),
        scratch_types=[
            pltpu.VMEM((N,), jnp.float32),
            pltpu.VMEM((N,), jnp.float32),
            pltpu.VMEM((N,), jnp.float32),
            pltpu.VMEM((N,), jnp.float32),
            pltpu.VMEM((N,), jnp.float32),
            pltpu.VMEM((N,), jnp.float32),
            pltpu.VMEM((N,), jnp.float32),
            pltpu.VMEM((N,), jnp.float32),
            pltpu.VMEM((N,), jnp.int32),
            pltpu.VMEM((GSH,), jnp.float32),
            pltpu.VMEM((GSH,), jnp.float32),
            pltpu.VMEM((GSH,), jnp.float32),
            pltpu.VMEM((GSH * 96,), jnp.float32),
        ],
        compiler_params=cp,
    )
    return kfn(x, y, z, cx, cy, cz)


def kernel(xyz, pose):
    B, N, _ = xyz.shape
    R = pose[:, :, :3]
    t = pose[:, :, 3]
    xyz_t = jnp.einsum('bij,bnj->bni', R, xyz) + t[:, None, :]
    planes = xyz_t.transpose(2, 0, 1)
    x, y, z = planes[0], planes[1], planes[2]

    cx, cy, cz = _fps_centers(x, y, z)
    center = jnp.stack([cx, cy, cz], axis=-1)

    flat = _knn_group_sc(x, y, z, cx, cy, cz)
    neighborhood = flat.reshape(B, NUM_GROUP, 3, GROUP_SIZE).transpose(0, 1, 3, 2)
    return (neighborhood, center)
